# Initial kernel scaffold; baseline (speedup 1.0000x reference)
import jax
import jax.numpy as jnp
from jax.experimental import pallas as pl


def kernel(drug_x, disease_x, dis_to_drug_src, dis_to_drug_dst, drug_to_dis_src, drug_to_dis_dst, W_drug, W_dis, a_drug, a_dis):
    raise NotImplementedError("write your pallas kernel here")


# speedup vs baseline: 4.4081x; 4.4081x over previous
import functools

import jax
import jax.numpy as jnp
from jax import lax
from jax.experimental import pallas as pl
from jax.experimental.pallas import tpu as pltpu
from jax.experimental.pallas import tpu_sc as plsc

N = 10000
E = 160000
D = 256
H = 128
NEG_SLOPE = 0.2
EPS = 1e-16

NC = 2
NS = 16
L = 16

EP = E // NS
EPP = 10240
K = 128
NCH = EPP // K
SEG = 624
WB = 48
NP = 10240
SEGP = NP // NS

_f32 = jnp.float32
_i32 = jnp.int32


def _tc_project(x, W, a_d, a_s):
    BM = 1000

    def body(x_ref, w_ref, ad_ref, as_ref, hlo_ref, hhi_ref, sd_ref, ss_ref):
        h = jnp.dot(x_ref[...], w_ref[...], preferred_element_type=_f32)
        hlo_ref[...] = h[:, :H]
        hhi_ref[...] = h[:, H:]
        sd_ref[...] = jnp.dot(h, ad_ref[...], preferred_element_type=_f32)
        ss_ref[...] = jnp.dot(h, as_ref[...], preferred_element_type=_f32)

    return pl.pallas_call(
        body,
        grid=(N // BM,),
        in_specs=[
            pl.BlockSpec((BM, D), lambda i: (i, 0)),
            pl.BlockSpec((D, D), lambda i: (0, 0)),
            pl.BlockSpec((D, 1), lambda i: (0, 0)),
            pl.BlockSpec((D, 1), lambda i: (0, 0)),
        ],
        out_specs=[
            pl.BlockSpec((BM, H), lambda i: (i, 0)),
            pl.BlockSpec((BM, H), lambda i: (i, 0)),
            pl.BlockSpec((BM, 1), lambda i: (i, 0)),
            pl.BlockSpec((BM, 1), lambda i: (i, 0)),
        ],
        out_shape=[
            jax.ShapeDtypeStruct((N, H), _f32),
            jax.ShapeDtypeStruct((N, H), _f32),
            jax.ShapeDtypeStruct((N, 1), _f32),
            jax.ShapeDtypeStruct((N, 1), _f32),
        ],
    )(x, W, a_d, a_s)


def _alpha_body(sdA, ssA, siA, diA, sdB, ssB, siB, diB,
                alA, alB,
                sv_src, sv_dst, isrc, idst, ebuf, dloc, red, dseg, mxbuf,
                dshr, dfin, mshr):
    c = lax.axis_index("c")
    s = lax.axis_index("s")
    z16 = jnp.zeros((L,), _f32)

    def run_dir(sd_h, ss_h, si_h, di_h, al_h):
        pltpu.sync_copy(sd_h, sv_dst)
        pltpu.sync_copy(ss_h, sv_src)
        pltpu.sync_copy(si_h.at[s], isrc)
        pltpu.sync_copy(di_h.at[s], idst)

        def zdl(g, _):
            dloc[pl.ds(g * L, L)] = z16
            return 0
        lax.fori_loop(0, NP // L, zdl, 0)

        def p1a(g, m):
            sl = pl.ds(g * L, L)
            e = (plsc.load_gather(sv_dst, [idst[sl]])
                 + plsc.load_gather(sv_src, [isrc[sl]]))
            e = jnp.where(e >= 0.0, e, e * NEG_SLOPE)
            ebuf[sl] = e
            return jnp.maximum(m, e)
        m = lax.fori_loop(0, EP // L, p1a, jnp.full((L,), -jnp.inf, _f32))

        mxbuf[0, pl.ds(0, L)] = m
        pltpu.sync_copy(mxbuf.at[0], mshr.at[s])
        plsc.subcore_barrier()
        pltpu.sync_copy(mshr, mxbuf)
        mv = mxbuf[0, pl.ds(0, L)]
        for t in range(1, NS):
            mv = jnp.maximum(mv, mxbuf[t, pl.ds(0, L)])
        M = jnp.max(mv)

        def p1b(g, _):
            sl = pl.ds(g * L, L)
            ex = jnp.exp(ebuf[sl] - M)
            ebuf[sl] = ex
            plsc.addupdate_scatter(dloc, [idst[sl]], ex)
            return 0
        lax.fori_loop(0, EP // L, p1b, 0)

        pltpu.sync_copy(dloc, dshr.at[s])
        plsc.subcore_barrier()
        for t in range(NS):
            pltpu.sync_copy(dshr.at[t, pl.ds(s * SEGP, SEGP)], red.at[t])

        def dred(g, _):
            sl = pl.ds(g * L, L)
            v = red[0, sl]
            for t in range(1, NS):
                v = v + red[t, sl]
            dseg[sl] = v
            return 0
        lax.fori_loop(0, SEGP // L, dred, 0)
        pltpu.sync_copy(dseg, dfin.at[pl.ds(s * SEGP, SEGP)])
        plsc.subcore_barrier()
        pltpu.sync_copy(dfin, dloc)

        def p1c(g, _):
            sl = pl.ds(g * L, L)
            dv = plsc.load_gather(dloc, [idst[sl]])
            ebuf[sl] = ebuf[sl] / (dv + EPS)
            return 0
        lax.fori_loop(0, EP // L, p1c, 0)
        pltpu.sync_copy(ebuf, al_h.at[s])

    @pl.when(c == 0)
    def _():
        run_dir(sdA, ssA, siA, diA, alA)

    @pl.when(c == 1)
    def _():
        run_dir(sdB, ssB, siB, diB, alB)


_sc_alpha = functools.partial(
    pl.kernel,
    out_type=[jax.ShapeDtypeStruct((NS, EP), _f32),
              jax.ShapeDtypeStruct((NS, EP), _f32)],
    mesh=plsc.VectorSubcoreMesh(core_axis_name="c", subcore_axis_name="s",
                                num_cores=NC, num_subcores=NS),
    compiler_params=pltpu.CompilerParams(needs_layout_passes=False),
    scratch_types=[
        pltpu.VMEM((N,), _f32),
        pltpu.VMEM((N,), _f32),
        pltpu.VMEM((EP,), _i32),
        pltpu.VMEM((EP,), _i32),
        pltpu.VMEM((EP,), _f32),
        pltpu.VMEM((NP,), _f32),
        pltpu.VMEM((NS, SEGP), _f32),
        pltpu.VMEM((SEGP,), _f32),
        pltpu.VMEM((NS, L), _f32),
        pltpu.VMEM_SHARED((NS, NP), _f32),
        pltpu.VMEM_SHARED((NP,), _f32),
        pltpu.VMEM_SHARED((NS, L), _f32),
    ],
)(_alpha_body)


def _agg_body(hA_lo, hA_hi, hB_lo, hB_hi, siA, diA, alA, siB, diB, alB,
              outA, outB,
              isrc, idst, abuf, rowbuf,
              acc):
    c = lax.axis_index("c")
    s = lax.axis_index("s")
    z16 = jnp.zeros((L,), _f32)

    def run_dir(tlo, thi, si_h, di_h, al_h, out_h):
        pltpu.sync_copy(si_h.at[s], isrc)
        pltpu.sync_copy(di_h.at[s], idst)

        def zrb(j, _):
            for f in range(H // L):
                rowbuf[j, pl.ds(f * L, L)] = z16
            return 0
        lax.fori_loop(0, WB, zrb, 0)
        zchunk = rowbuf.at[pl.ds(0, WB)]
        for q in range(SEG // WB):
            pltpu.sync_copy(zchunk, acc.at[pl.ds(s * SEG + q * WB, WB)])

        @pl.when(s == NS - 1)
        def _():
            pltpu.sync_copy(rowbuf.at[pl.ds(0, 16)],
                            acc.at[pl.ds(NS * SEG, 16)])
        plsc.subcore_barrier()

        def phase2(tbl):
            def p2(j, _):
                pltpu.sync_copy(al_h.at[s, j], abuf)
                pltpu.sync_copy(tbl.at[isrc.at[j]], rowbuf)

                def scale(i, _):
                    av = plsc.load_gather(abuf, [jnp.full((L,), i, _i32)])
                    for f in range(H // L):
                        sl = pl.ds(f * L, L)
                        rowbuf[i, sl] = rowbuf[i, sl] * av
                    return 0
                lax.fori_loop(0, K, scale, 0)
                pltpu.sync_copy(rowbuf, acc.at[idst.at[j]], add=True)
                return 0
            lax.fori_loop(0, NCH, p2, 0)

        @pl.when(c == 0)
        def _():
            phase2(tlo)

        @pl.when(c == 1)
        def _():
            phase2(thi)
        plsc.subcore_barrier()

        def write_rows(r0, nrows):
            pltpu.sync_copy(acc.at[pl.ds(r0, nrows)],
                            rowbuf.at[pl.ds(0, nrows)])

            def wrelu(j, _):
                for f in range(H // L):
                    sl = pl.ds(f * L, L)
                    rowbuf[j, sl] = jnp.maximum(rowbuf[j, sl], 0.0)
                return 0
            lax.fori_loop(0, nrows, wrelu, 0)

            @pl.when(c == 0)
            def _():
                pltpu.sync_copy(rowbuf.at[pl.ds(0, nrows)],
                                out_h.at[pl.ds(r0, nrows), pl.ds(0, H)])

            @pl.when(c == 1)
            def _():
                pltpu.sync_copy(rowbuf.at[pl.ds(0, nrows)],
                                out_h.at[pl.ds(r0, nrows), pl.ds(H, H)])

        for q in range(SEG // WB):
            write_rows(s * SEG + q * WB, WB)

        @pl.when(s == NS - 1)
        def _():
            write_rows(NS * SEG, 16)
        plsc.subcore_barrier()

    run_dir(hA_lo, hA_hi, siA, diA, alA, outA)
    run_dir(hB_lo, hB_hi, siB, diB, alB, outB)


_sc_agg = functools.partial(
    pl.kernel,
    out_type=[jax.ShapeDtypeStruct((N, D), _f32),
              jax.ShapeDtypeStruct((N, D), _f32)],
    mesh=plsc.VectorSubcoreMesh(core_axis_name="c", subcore_axis_name="s",
                                num_cores=NC, num_subcores=NS),
    compiler_params=pltpu.CompilerParams(needs_layout_passes=False),
    scratch_types=[
        pltpu.VMEM((NCH, K), _i32),
        pltpu.VMEM((NCH, K), _i32),
        pltpu.VMEM((K,), _f32),
        pltpu.VMEM((K, H), _f32),
        pltpu.VMEM_SHARED((N, H), _f32),
    ],
)(_agg_body)


@jax.jit
def kernel(drug_x, disease_x, dis_to_drug_src, dis_to_drug_dst,
           drug_to_dis_src, drug_to_dis_dst, W_drug, W_dis, a_drug, a_dis):
    hdr_lo, hdr_hi, sdA, ssB = _tc_project(drug_x, W_drug,
                                           a_drug[:D], a_dis[D:])
    hds_lo, hds_hi, sdB, ssA = _tc_project(disease_x, W_dis,
                                           a_dis[:D], a_drug[D:])

    def _pad3(a):
        return jnp.pad(a, ((0, 0), (0, EPP - EP))).reshape(NS, NCH, K)

    siA = dis_to_drug_src.astype(_i32).reshape(NS, EP)
    diA = dis_to_drug_dst.astype(_i32).reshape(NS, EP)
    siB = drug_to_dis_src.astype(_i32).reshape(NS, EP)
    diB = drug_to_dis_dst.astype(_i32).reshape(NS, EP)

    alA, alB = _sc_alpha(sdA.reshape(N), ssA.reshape(N), siA, diA,
                         sdB.reshape(N), ssB.reshape(N), siB, diB)

    drug_out, dis_out = _sc_agg(hds_lo, hds_hi, hdr_lo, hdr_hi,
                                _pad3(siA), _pad3(diA), _pad3(alA),
                                _pad3(siB), _pad3(diB), _pad3(alB))
    return (drug_out, dis_out)


# speedup vs baseline: 5.9820x; 1.3570x over previous
import functools

import jax
import jax.numpy as jnp
from jax import lax
from jax.experimental import pallas as pl
from jax.experimental.pallas import tpu as pltpu
from jax.experimental.pallas import tpu_sc as plsc

N = 10000
E = 160000
D = 256
H = 128
NEG_SLOPE = 0.2
EPS = 1e-16

NC = 2
NS = 16
L = 16

EP = E // NS
EPP = 10240
K = 128
NCH = EPP // K
SEG = 624
WB = 48
NP = 10240
SEGP = NP // NS

_f32 = jnp.float32
_i32 = jnp.int32


def _tc_project(x, W, a_d, a_s):
    BM = 1000

    def body(x_ref, w_ref, ad_ref, as_ref, hlo_ref, hhi_ref, sd_ref, ss_ref):
        h = jnp.dot(x_ref[...], w_ref[...], preferred_element_type=_f32)
        hlo_ref[...] = h[:, :H]
        hhi_ref[...] = h[:, H:]
        sd_ref[...] = jnp.dot(h, ad_ref[...], preferred_element_type=_f32)
        ss_ref[...] = jnp.dot(h, as_ref[...], preferred_element_type=_f32)

    return pl.pallas_call(
        body,
        grid=(N // BM,),
        in_specs=[
            pl.BlockSpec((BM, D), lambda i: (i, 0)),
            pl.BlockSpec((D, D), lambda i: (0, 0)),
            pl.BlockSpec((D, 1), lambda i: (0, 0)),
            pl.BlockSpec((D, 1), lambda i: (0, 0)),
        ],
        out_specs=[
            pl.BlockSpec((BM, H), lambda i: (i, 0)),
            pl.BlockSpec((BM, H), lambda i: (i, 0)),
            pl.BlockSpec((BM, 1), lambda i: (i, 0)),
            pl.BlockSpec((BM, 1), lambda i: (i, 0)),
        ],
        out_shape=[
            jax.ShapeDtypeStruct((N, H), _f32),
            jax.ShapeDtypeStruct((N, H), _f32),
            jax.ShapeDtypeStruct((N, 1), _f32),
            jax.ShapeDtypeStruct((N, 1), _f32),
        ],
    )(x, W, a_d, a_s)


def _alpha_body(sdA, ssA, siA, diA, sdB, ssB, siB, diB,
                alA, alB,
                sv_src, sv_dst, isrc, idst, ebuf, dloc, red, dseg, mxbuf,
                dshr, dfin, mshr):
    c = lax.axis_index("c")
    s = lax.axis_index("s")
    z16 = jnp.zeros((L,), _f32)

    def run_dir(sd_h, ss_h, si_h, di_h, al_h):
        pltpu.sync_copy(sd_h, sv_dst)
        pltpu.sync_copy(ss_h, sv_src)
        pltpu.sync_copy(si_h.at[s], isrc)
        pltpu.sync_copy(di_h.at[s], idst)

        def zdl(g, _):
            dloc[pl.ds(g * L, L)] = z16
            return 0
        lax.fori_loop(0, NP // L, zdl, 0)

        def p1a(g, m):
            sl = pl.ds(g * L, L)
            e = (plsc.load_gather(sv_dst, [idst[sl]])
                 + plsc.load_gather(sv_src, [isrc[sl]]))
            e = jnp.where(e >= 0.0, e, e * NEG_SLOPE)
            ebuf[sl] = e
            return jnp.maximum(m, e)
        m = lax.fori_loop(0, EP // L, p1a, jnp.full((L,), -jnp.inf, _f32))

        mxbuf[0, pl.ds(0, L)] = m
        pltpu.sync_copy(mxbuf.at[0], mshr.at[s])
        plsc.subcore_barrier()
        pltpu.sync_copy(mshr, mxbuf)
        mv = mxbuf[0, pl.ds(0, L)]
        for t in range(1, NS):
            mv = jnp.maximum(mv, mxbuf[t, pl.ds(0, L)])
        M = jnp.max(mv)

        def p1b(g, _):
            sl = pl.ds(g * L, L)
            ex = jnp.exp(ebuf[sl] - M)
            ebuf[sl] = ex
            plsc.addupdate_scatter(dloc, [idst[sl]], ex)
            return 0
        lax.fori_loop(0, EP // L, p1b, 0)

        pltpu.sync_copy(dloc, dshr.at[s])
        plsc.subcore_barrier()
        for t in range(NS):
            pltpu.sync_copy(dshr.at[t, pl.ds(s * SEGP, SEGP)], red.at[t])

        def dred(g, _):
            sl = pl.ds(g * L, L)
            v = red[0, sl]
            for t in range(1, NS):
                v = v + red[t, sl]
            dseg[sl] = v
            return 0
        lax.fori_loop(0, SEGP // L, dred, 0)
        pltpu.sync_copy(dseg, dfin.at[pl.ds(s * SEGP, SEGP)])
        plsc.subcore_barrier()
        pltpu.sync_copy(dfin, dloc)

        def p1c(g, _):
            sl = pl.ds(g * L, L)
            dv = plsc.load_gather(dloc, [idst[sl]])
            ebuf[sl] = ebuf[sl] / (dv + EPS)
            return 0
        lax.fori_loop(0, EP // L, p1c, 0)
        pltpu.sync_copy(ebuf, al_h.at[s])

    @pl.when(c == 0)
    def _():
        run_dir(sdA, ssA, siA, diA, alA)

    @pl.when(c == 1)
    def _():
        run_dir(sdB, ssB, siB, diB, alB)


_sc_alpha = functools.partial(
    pl.kernel,
    out_type=[jax.ShapeDtypeStruct((NS, EP), _f32),
              jax.ShapeDtypeStruct((NS, EP), _f32)],
    mesh=plsc.VectorSubcoreMesh(core_axis_name="c", subcore_axis_name="s",
                                num_cores=NC, num_subcores=NS),
    compiler_params=pltpu.CompilerParams(needs_layout_passes=False),
    scratch_types=[
        pltpu.VMEM((N,), _f32),
        pltpu.VMEM((N,), _f32),
        pltpu.VMEM((EP,), _i32),
        pltpu.VMEM((EP,), _i32),
        pltpu.VMEM((EP,), _f32),
        pltpu.VMEM((NP,), _f32),
        pltpu.VMEM((NS, SEGP), _f32),
        pltpu.VMEM((SEGP,), _f32),
        pltpu.VMEM((NS, L), _f32),
        pltpu.VMEM_SHARED((NS, NP), _f32),
        pltpu.VMEM_SHARED((NP,), _f32),
        pltpu.VMEM_SHARED((NS, L), _f32),
    ],
)(_alpha_body)


def _agg_body(hA_lo, hA_hi, hB_lo, hB_hi, siA, diA, alA, siB, diB, alB,
              outA, outB,
              isrc, ibuf, abuf, rowbuf, sem0, sem1,
              acc):
    c = lax.axis_index("c")
    s = lax.axis_index("s")
    z16 = jnp.zeros((L,), _f32)
    sems = (sem0, sem1)

    def run_dir(tlo, thi, si_h, di_h, al_h, out_h):
        pltpu.sync_copy(si_h.at[s], isrc)

        def zrb(j, _):
            for f in range(H // L):
                rowbuf[0, j, pl.ds(f * L, L)] = z16
            return 0
        lax.fori_loop(0, WB, zrb, 0)
        zchunk = rowbuf.at[0, pl.ds(0, WB)]
        for q in range(SEG // WB):
            pltpu.sync_copy(zchunk, acc.at[pl.ds(s * SEG + q * WB, WB)])

        @pl.when(s == NS - 1)
        def _():
            pltpu.sync_copy(rowbuf.at[0, pl.ds(0, 16)],
                            acc.at[pl.ds(NS * SEG, 16)])
        plsc.subcore_barrier()

        def phase2(tbl):
            def start_in(j, b):
                pltpu.async_copy(di_h.at[s, j], ibuf.at[b], sems[b])
                pltpu.async_copy(al_h.at[s, j], abuf.at[b], sems[b])
                pltpu.async_copy(tbl.at[isrc.at[j]], rowbuf.at[b], sems[b])

            def wait_in(j, b):
                pltpu.make_async_copy(di_h.at[s, j], ibuf.at[b],
                                      sems[b]).wait()
                pltpu.make_async_copy(al_h.at[s, j], abuf.at[b],
                                      sems[b]).wait()
                pltpu.make_async_copy(tbl.at[isrc.at[j]], rowbuf.at[b],
                                      sems[b]).wait()

            def work(j, b):
                wait_in(j, b)

                @pl.when(j + 1 < NCH)
                def _():
                    start_in(j + 1, 1 - b)

                def scale(i, _):
                    av = plsc.load_gather(
                        abuf, [jnp.full((L,), b, _i32),
                               jnp.full((L,), i, _i32)])
                    for f in range(H // L):
                        sl = pl.ds(f * L, L)
                        rowbuf[b, i, sl] = rowbuf[b, i, sl] * av
                    return 0
                lax.fori_loop(0, K, scale, 0)
                pltpu.sync_copy(rowbuf.at[b], acc.at[ibuf.at[b]], add=True)

            start_in(0, 0)

            def p2(t, _):
                work(2 * t, 0)
                work(2 * t + 1, 1)
                return 0
            lax.fori_loop(0, NCH // 2, p2, 0)

        @pl.when(c == 0)
        def _():
            phase2(tlo)

        @pl.when(c == 1)
        def _():
            phase2(thi)
        plsc.subcore_barrier()

        def write_rows(r0, nrows):
            stg = rowbuf.at[0, pl.ds(0, nrows)]
            pltpu.sync_copy(acc.at[pl.ds(r0, nrows)], stg)

            def wrelu(j, _):
                for f in range(H // L):
                    sl = pl.ds(f * L, L)
                    rowbuf[0, j, sl] = jnp.maximum(rowbuf[0, j, sl], 0.0)
                return 0
            lax.fori_loop(0, nrows, wrelu, 0)

            @pl.when(c == 0)
            def _():
                pltpu.sync_copy(stg, out_h.at[pl.ds(r0, nrows), pl.ds(0, H)])

            @pl.when(c == 1)
            def _():
                pltpu.sync_copy(stg, out_h.at[pl.ds(r0, nrows), pl.ds(H, H)])

        for q in range(SEG // WB):
            write_rows(s * SEG + q * WB, WB)

        @pl.when(s == NS - 1)
        def _():
            write_rows(NS * SEG, 16)
        plsc.subcore_barrier()

    run_dir(hA_lo, hA_hi, siA, diA, alA, outA)
    run_dir(hB_lo, hB_hi, siB, diB, alB, outB)


_sc_agg = functools.partial(
    pl.kernel,
    out_type=[jax.ShapeDtypeStruct((N, D), _f32),
              jax.ShapeDtypeStruct((N, D), _f32)],
    mesh=plsc.VectorSubcoreMesh(core_axis_name="c", subcore_axis_name="s",
                                num_cores=NC, num_subcores=NS),
    compiler_params=pltpu.CompilerParams(needs_layout_passes=False),
    scratch_types=[
        pltpu.VMEM((NCH, K), _i32),
        pltpu.VMEM((2, K), _i32),
        pltpu.VMEM((2, K), _f32),
        pltpu.VMEM((2, K, H), _f32),
        pltpu.SemaphoreType.DMA,
        pltpu.SemaphoreType.DMA,
        pltpu.VMEM_SHARED((N, H), _f32),
    ],
)(_agg_body)


@jax.jit
def kernel(drug_x, disease_x, dis_to_drug_src, dis_to_drug_dst,
           drug_to_dis_src, drug_to_dis_dst, W_drug, W_dis, a_drug, a_dis):
    hdr_lo, hdr_hi, sdA, ssB = _tc_project(drug_x, W_drug,
                                           a_drug[:D], a_dis[D:])
    hds_lo, hds_hi, sdB, ssA = _tc_project(disease_x, W_dis,
                                           a_dis[:D], a_drug[D:])

    def _pad3(a):
        return jnp.pad(a, ((0, 0), (0, EPP - EP))).reshape(NS, NCH, K)

    siA = dis_to_drug_src.astype(_i32).reshape(NS, EP)
    diA = dis_to_drug_dst.astype(_i32).reshape(NS, EP)
    siB = drug_to_dis_src.astype(_i32).reshape(NS, EP)
    diB = drug_to_dis_dst.astype(_i32).reshape(NS, EP)

    alA, alB = _sc_alpha(sdA.reshape(N), ssA.reshape(N), siA, diA,
                         sdB.reshape(N), ssB.reshape(N), siB, diB)

    drug_out, dis_out = _sc_agg(hds_lo, hds_hi, hdr_lo, hdr_hi,
                                _pad3(siA), _pad3(diA), _pad3(alA),
                                _pad3(siB), _pad3(diB), _pad3(alB))
    return (drug_out, dis_out)
